# Initial kernel scaffold; baseline (speedup 1.0000x reference)
import jax
import jax.numpy as jnp
from jax.experimental import pallas as pl


def kernel(pair_features, atom_features, atom_to_pair, W, b):
    raise NotImplementedError("write your pallas kernel here")


# speedup vs baseline: 4.3436x; 4.3436x over previous
import jax
import jax.numpy as jnp
from jax import lax
from jax.experimental import pallas as pl
from jax.experimental.pallas import tpu as pltpu
from jax.experimental.pallas import tpu_sc as plsc

N_NODES = 10000
N_EDGES = 50000
N_PAIR = 16
H = 64

NC = 2
NS = 16
NW = NC * NS
EDGES_PAD = 50176
EPW = EDGES_PAD // NW
GCH = 112
GITERS = EPW // GCH

SPW = EDGES_PAD // NS
SITERS = SPW // GCH
ACC_ROWS = 10016
ZROWS = ACC_ROWS // NS
TROWS = N_NODES // NS

_mesh = plsc.VectorSubcoreMesh(core_axis_name="c", subcore_axis_name="s")


def _gather_body(af, src, l_out, idx_v, rows_v, sem):
    c = lax.axis_index("c")
    s = lax.axis_index("s")
    base = (s * NC + c) * EPW
    pltpu.sync_copy(src.at[pl.ds(base, EPW)], idx_v)

    def step(i, carry):
        pltpu.async_copy(af.at[idx_v.at[pl.ds(i * GCH, GCH)]], rows_v, sem).wait()
        pltpu.sync_copy(rows_v, l_out.at[pl.ds(base + i * GCH, GCH)])
        return carry

    lax.fori_loop(0, GITERS, step, 0)


_gather = pl.kernel(
    _gather_body,
    out_type=jax.ShapeDtypeStruct((EDGES_PAD, H), jnp.float32),
    mesh=_mesh,
    compiler_params=pltpu.CompilerParams(use_tc_tiling_on_sc=False),
    scratch_types=[
        pltpu.VMEM((EPW,), jnp.int32),
        pltpu.VMEM((GCH, H), jnp.float32),
        pltpu.SemaphoreType.DMA,
    ],
)


def _scatter_body(out_e, dst, zrows, t_out, idx_v, rows_v, acc):
    c = lax.axis_index("c")
    s = lax.axis_index("s")

    @pl.when(c == 0)
    def _():
        pltpu.sync_copy(zrows, acc.at[pl.ds(s * ZROWS, ZROWS)])
        plsc.subcore_barrier()
        base = s * SPW

        def step(i, carry):
            pltpu.sync_copy(dst.at[pl.ds(base + i * GCH, GCH)], idx_v)
            pltpu.sync_copy(out_e.at[pl.ds(base + i * GCH, GCH)], rows_v)
            pltpu.sync_copy(rows_v, acc.at[idx_v], add=True)
            return carry

        lax.fori_loop(0, SITERS, step, 0)
        plsc.subcore_barrier()
        pltpu.sync_copy(acc.at[pl.ds(s * TROWS, TROWS)],
                        t_out.at[pl.ds(s * TROWS, TROWS)])


_scatter = pl.kernel(
    _scatter_body,
    out_type=jax.ShapeDtypeStruct((N_NODES, H), jnp.float32),
    mesh=_mesh,
    compiler_params=pltpu.CompilerParams(use_tc_tiling_on_sc=False),
    scratch_types=[
        pltpu.VMEM((GCH,), jnp.int32),
        pltpu.VMEM((GCH, H), jnp.float32),
        pltpu.VMEM_SHARED((ACC_ROWS, H), jnp.float32),
    ],
)

TILE = 1024
GRID = EDGES_PAD // TILE


def _mm_body(pf_ref, l_ref, w_ref, out_ref):
    lv = l_ref[...]
    pieces = [lv]
    for p in range(N_PAIR):
        pieces.append(pf_ref[:, p:p + 1] * lv)
    g = jnp.concatenate(pieces, axis=1)
    out_ref[...] = jnp.dot(g, w_ref[...], preferred_element_type=jnp.float32)


_mm = pl.pallas_call(
    _mm_body,
    grid=(GRID,),
    in_specs=[
        pl.BlockSpec((TILE, N_PAIR), lambda i: (i, 0)),
        pl.BlockSpec((TILE, H), lambda i: (i, 0)),
        pl.BlockSpec(((N_PAIR + 1) * H, H), lambda i: (0, 0)),
    ],
    out_specs=pl.BlockSpec((TILE, H), lambda i: (i, 0)),
    out_shape=jax.ShapeDtypeStruct((EDGES_PAD, H), jnp.float32),
)


def kernel(pair_features, atom_features, atom_to_pair, W, b):
    pad = EDGES_PAD - N_EDGES
    src_p = jnp.concatenate([atom_to_pair[:, 1], jnp.zeros((pad,), jnp.int32)])
    dst_p = jnp.concatenate(
        [atom_to_pair[:, 0], jnp.full((pad,), N_NODES, jnp.int32)])
    pf_p = jnp.concatenate(
        [pair_features, jnp.zeros((pad, N_PAIR), jnp.float32)])
    wbig = jnp.concatenate(
        [b.reshape(1, H, H), W.reshape(N_PAIR, H, H)], axis=0)
    wbig = wbig.transpose(0, 2, 1).reshape((N_PAIR + 1) * H, H)

    l = _gather(atom_features, src_p)
    out_e = _mm(pf_p, l, wbig)
    zrows = jnp.zeros((ZROWS, H), jnp.float32)
    return _scatter(out_e, dst_p, zrows)


# speedup vs baseline: 5.0412x; 1.1606x over previous
import jax
import jax.numpy as jnp
from jax import lax
from jax.experimental import pallas as pl
from jax.experimental.pallas import tpu as pltpu
from jax.experimental.pallas import tpu_sc as plsc

N_NODES = 10000
N_EDGES = 50000
N_PAIR = 16
H = 64

NC = 2
NS = 16
NW = NC * NS
EDGES_PAD = 50176
EPW = EDGES_PAD // NW
GCH = 112
GITERS = EPW // GCH

SPW = EDGES_PAD // NS
SITERS = SPW // GCH
ACC_ROWS = 10016
ZROWS = ACC_ROWS // NS
TROWS = N_NODES // NS

_mesh = plsc.VectorSubcoreMesh(core_axis_name="c", subcore_axis_name="s")


def _gather_body(af, src, l_out, idx_v, rows_v, sem):
    c = lax.axis_index("c")
    s = lax.axis_index("s")
    base = (s * NC + c) * EPW
    pltpu.sync_copy(src.at[pl.ds(base, EPW)], idx_v)

    def step(i, carry):
        pltpu.async_copy(af.at[idx_v.at[pl.ds(i * GCH, GCH)]], rows_v, sem).wait()
        pltpu.sync_copy(rows_v, l_out.at[pl.ds(base + i * GCH, GCH)])
        return carry

    lax.fori_loop(0, GITERS, step, 0)


_gather = pl.kernel(
    _gather_body,
    out_type=jax.ShapeDtypeStruct((EDGES_PAD, H), jnp.float32),
    mesh=_mesh,
    compiler_params=pltpu.CompilerParams(use_tc_tiling_on_sc=False),
    scratch_types=[
        pltpu.VMEM((EPW,), jnp.int32),
        pltpu.VMEM((GCH, H), jnp.float32),
        pltpu.SemaphoreType.DMA,
    ],
)


def _scatter_body(out_e, dst, zrows, t_out, idx_v, rows_v, acc):
    c = lax.axis_index("c")
    s = lax.axis_index("s")

    @pl.when(c == 0)
    def _():
        pltpu.sync_copy(zrows, acc.at[pl.ds(s * ZROWS, ZROWS)])
        plsc.subcore_barrier()
        base = s * SPW

        def step(i, carry):
            pltpu.sync_copy(dst.at[pl.ds(base + i * GCH, GCH)], idx_v)
            pltpu.sync_copy(out_e.at[pl.ds(base + i * GCH, GCH)], rows_v)
            pltpu.sync_copy(rows_v, acc.at[idx_v], add=True)
            return carry

        lax.fori_loop(0, SITERS, step, 0)
        plsc.subcore_barrier()
        pltpu.sync_copy(acc.at[pl.ds(s * TROWS, TROWS)],
                        t_out.at[pl.ds(s * TROWS, TROWS)])


_scatter = pl.kernel(
    _scatter_body,
    out_type=jax.ShapeDtypeStruct((N_NODES, H), jnp.float32),
    mesh=_mesh,
    compiler_params=pltpu.CompilerParams(use_tc_tiling_on_sc=False),
    scratch_types=[
        pltpu.VMEM((GCH,), jnp.int32),
        pltpu.VMEM((GCH, H), jnp.float32),
        pltpu.VMEM_SHARED((ACC_ROWS, H), jnp.float32),
    ],
)

TILE = 1024
GRID = EDGES_PAD // TILE


def _mm_body(pf_ref, l_ref, w_ref, out_ref):
    lv = l_ref[...]
    acc = jnp.dot(lv, w_ref[0], preferred_element_type=jnp.float32)
    for p in range(N_PAIR):
        acc += pf_ref[:, p:p + 1] * jnp.dot(
            lv, w_ref[p + 1], preferred_element_type=jnp.float32)
    out_ref[...] = acc


_mm = pl.pallas_call(
    _mm_body,
    grid=(GRID,),
    in_specs=[
        pl.BlockSpec((TILE, N_PAIR), lambda i: (i, 0)),
        pl.BlockSpec((TILE, H), lambda i: (i, 0)),
        pl.BlockSpec((N_PAIR + 1, H, H), lambda i: (0, 0, 0)),
    ],
    out_specs=pl.BlockSpec((TILE, H), lambda i: (i, 0)),
    out_shape=jax.ShapeDtypeStruct((EDGES_PAD, H), jnp.float32),
)


def kernel(pair_features, atom_features, atom_to_pair, W, b):
    pad = EDGES_PAD - N_EDGES
    src_p = jnp.concatenate([atom_to_pair[:, 1], jnp.zeros((pad,), jnp.int32)])
    dst_p = jnp.concatenate(
        [atom_to_pair[:, 0], jnp.full((pad,), N_NODES, jnp.int32)])
    pf_p = jnp.concatenate(
        [pair_features, jnp.zeros((pad, N_PAIR), jnp.float32)])
    wbig = jnp.concatenate(
        [b.reshape(1, H, H), W.reshape(N_PAIR, H, H)], axis=0)
    wbig = wbig.transpose(0, 2, 1)

    l = _gather(atom_features, src_p)
    out_e = _mm(pf_p, l, wbig)
    zrows = jnp.zeros((ZROWS, H), jnp.float32)
    return _scatter(out_e, dst_p, zrows)


# speedup vs baseline: 5.3295x; 1.0572x over previous
import jax
import jax.numpy as jnp
from jax import lax
from jax.experimental import pallas as pl
from jax.experimental.pallas import tpu as pltpu
from jax.experimental.pallas import tpu_sc as plsc

N_NODES = 10000
N_EDGES = 50000
N_PAIR = 16
H = 64

NC = 2
NS = 16
NW = NC * NS
EDGES_PAD = 50176
EPW = EDGES_PAD // NW
GCH = 112
CHUNKS = EPW // GCH
NBUF = 4

HALF = N_NODES // NC
ACC_ROWS = 5120
ZROWS = ACC_ROWS // NS
DUMMY = HALF
TROWS = 625
SPS = EDGES_PAD // NS
SCHUNKS = SPS // GCH

_mesh = plsc.VectorSubcoreMesh(core_axis_name="c", subcore_axis_name="s")
_params = pltpu.CompilerParams(use_tc_tiling_on_sc=False)


def _gather_body(af, src, l_out, idx_v, b0, b1, b2, b3, s0, s1, s2, s3):
    c = lax.axis_index("c")
    s = lax.axis_index("s")
    base = (s * NC + c) * EPW
    pltpu.sync_copy(src.at[pl.ds(base, EPW)], idx_v)
    bufs = (b0, b1, b2, b3)
    sems = (s0, s1, s2, s3)

    def g(j):
        return pltpu.async_copy(
            af.at[idx_v.at[pl.ds(j * GCH, GCH)]], bufs[j % NBUF], sems[j % NBUF])

    gd = [None] * CHUNKS
    wd = [None] * CHUNKS
    for j in range(NBUF - 1):
        gd[j] = g(j)
    for j in range(CHUNKS):
        gd[j].wait()
        if j > 0:
            wd[j - 1].wait()
        wd[j] = pltpu.async_copy(
            bufs[j % NBUF], l_out.at[pl.ds(base + j * GCH, GCH)], sems[j % NBUF])
        if j + NBUF - 1 < CHUNKS:
            gd[j + NBUF - 1] = g(j + NBUF - 1)
    wd[CHUNKS - 1].wait()


_gather = pl.kernel(
    _gather_body,
    out_type=jax.ShapeDtypeStruct((EDGES_PAD, H), jnp.float32),
    mesh=_mesh,
    compiler_params=_params,
    scratch_types=[
        pltpu.VMEM((EPW,), jnp.int32),
        pltpu.VMEM((GCH, H), jnp.float32),
        pltpu.VMEM((GCH, H), jnp.float32),
        pltpu.VMEM((GCH, H), jnp.float32),
        pltpu.VMEM((GCH, H), jnp.float32),
        pltpu.SemaphoreType.DMA,
        pltpu.SemaphoreType.DMA,
        pltpu.SemaphoreType.DMA,
        pltpu.SemaphoreType.DMA,
    ],
)


def _scatter_body(out_e, dsta, zrows, t_out, i0, i1, r0, r1, acc,
                  s0, s1, si0, si1):
    c = lax.axis_index("c")
    s = lax.axis_index("s")
    pltpu.sync_copy(zrows, acc.at[pl.ds(s * ZROWS, ZROWS)])
    plsc.subcore_barrier()

    base = s * SPS
    ibase = c * EDGES_PAD + base
    ibufs = (i0, i1)
    rbufs = (r0, r1)
    rsems = (s0, s1)
    isems = (si0, si1)
    rd = [None] * SCHUNKS
    idd = [None] * SCHUNKS

    def load(j):
        b = j % 2
        rd[j] = pltpu.async_copy(
            out_e.at[pl.ds(base + j * GCH, GCH)], rbufs[b], rsems[b])
        idd[j] = pltpu.async_copy(
            dsta.at[pl.ds(ibase + j * GCH, GCH)], ibufs[b], isems[b])

    load(0)
    for j in range(SCHUNKS):
        rd[j].wait()
        idd[j].wait()
        if j + 1 < SCHUNKS:
            load(j + 1)
        pltpu.sync_copy(rbufs[j % 2], acc.at[ibufs[j % 2]], add=True)
    plsc.subcore_barrier()

    @pl.when(s < 8)
    def _():
        pltpu.sync_copy(acc.at[pl.ds(s * TROWS, TROWS)],
                        t_out.at[pl.ds(c * HALF + s * TROWS, TROWS)])


_scatter = pl.kernel(
    _scatter_body,
    out_type=jax.ShapeDtypeStruct((N_NODES, H), jnp.float32),
    mesh=_mesh,
    compiler_params=_params,
    scratch_types=[
        pltpu.VMEM((GCH,), jnp.int32),
        pltpu.VMEM((GCH,), jnp.int32),
        pltpu.VMEM((GCH, H), jnp.float32),
        pltpu.VMEM((GCH, H), jnp.float32),
        pltpu.VMEM_SHARED((ACC_ROWS, H), jnp.float32),
        pltpu.SemaphoreType.DMA,
        pltpu.SemaphoreType.DMA,
        pltpu.SemaphoreType.DMA,
        pltpu.SemaphoreType.DMA,
    ],
)

TILE = 1024
GRID = EDGES_PAD // TILE


def _mm_body(pf_ref, l_ref, w_ref, out_ref):
    lv = l_ref[...]
    acc = jnp.dot(lv, w_ref[0], preferred_element_type=jnp.float32)
    for p in range(N_PAIR):
        acc += pf_ref[:, p:p + 1] * jnp.dot(
            lv, w_ref[p + 1], preferred_element_type=jnp.float32)
    out_ref[...] = acc


_mm = pl.pallas_call(
    _mm_body,
    grid=(GRID,),
    in_specs=[
        pl.BlockSpec((TILE, N_PAIR), lambda i: (i, 0)),
        pl.BlockSpec((TILE, H), lambda i: (i, 0)),
        pl.BlockSpec((N_PAIR + 1, H, H), lambda i: (0, 0, 0)),
    ],
    out_specs=pl.BlockSpec((TILE, H), lambda i: (i, 0)),
    out_shape=jax.ShapeDtypeStruct((EDGES_PAD, H), jnp.float32),
)


def kernel(pair_features, atom_features, atom_to_pair, W, b):
    pad = EDGES_PAD - N_EDGES
    src_p = jnp.concatenate([atom_to_pair[:, 1], jnp.zeros((pad,), jnp.int32)])
    dst_p = jnp.concatenate(
        [atom_to_pair[:, 0], jnp.full((pad,), N_NODES, jnp.int32)])
    halves = jnp.arange(NC, dtype=jnp.int32)[:, None] * HALF
    dadj = dst_p[None] - halves
    dadj = jnp.where((dadj >= 0) & (dadj < HALF), dadj, DUMMY)
    dadj = dadj.reshape(NC * EDGES_PAD)
    pf_p = jnp.concatenate(
        [pair_features, jnp.zeros((pad, N_PAIR), jnp.float32)])
    wbig = jnp.concatenate(
        [b.reshape(1, H, H), W.reshape(N_PAIR, H, H)], axis=0)
    wbig = wbig.transpose(0, 2, 1)

    l = _gather(atom_features, src_p)
    out_e = _mm(pf_p, l, wbig)
    zrows = jnp.zeros((ZROWS, H), jnp.float32)
    return _scatter(out_e, dadj, zrows)


# speedup vs baseline: 6.1138x; 1.1472x over previous
import jax
import jax.numpy as jnp
from jax import lax
from jax.experimental import pallas as pl
from jax.experimental.pallas import tpu as pltpu
from jax.experimental.pallas import tpu_sc as plsc

N_NODES = 10000
N_EDGES = 50000
N_PAIR = 16
H = 64
HW = 128

NC = 2
NS = 16
NW = NC * NS
EDGES_PAD = 50176
EPW = EDGES_PAD // NW
GCH = 112
CHUNKS = EPW // GCH
NBUF = 4

HALF = N_NODES // NC
ACC_ROWS = 5120
ZROWS = ACC_ROWS // NS
DUMMY = HALF
TROWS = 625
SPS = EDGES_PAD // NS
SCHUNKS = SPS // GCH

_mesh = plsc.VectorSubcoreMesh(core_axis_name="c", subcore_axis_name="s")
_params = pltpu.CompilerParams(use_tc_tiling_on_sc=False)


def _gather_body(af, src, l_out, idx_v, b0, b1, b2, b3, s0, s1, s2, s3):
    c = lax.axis_index("c")
    s = lax.axis_index("s")
    base = (s * NC + c) * EPW
    pltpu.sync_copy(src.at[pl.ds(base, EPW)], idx_v)
    bufs = (b0, b1, b2, b3)
    sems = (s0, s1, s2, s3)

    def g(j):
        return pltpu.async_copy(
            af.at[idx_v.at[pl.ds(j * GCH, GCH)]], bufs[j % NBUF], sems[j % NBUF])

    gd = [None] * CHUNKS
    wd = [None] * CHUNKS
    for j in range(NBUF - 1):
        gd[j] = g(j)
    for j in range(CHUNKS):
        gd[j].wait()
        if j > 0:
            wd[j - 1].wait()
        wd[j] = pltpu.async_copy(
            bufs[j % NBUF], l_out.at[pl.ds(base + j * GCH, GCH)], sems[j % NBUF])
        if j + NBUF - 1 < CHUNKS:
            gd[j + NBUF - 1] = g(j + NBUF - 1)
    wd[CHUNKS - 1].wait()


_gather = pl.kernel(
    _gather_body,
    out_type=jax.ShapeDtypeStruct((EDGES_PAD, HW), jnp.float32),
    mesh=_mesh,
    compiler_params=_params,
    scratch_types=[
        pltpu.VMEM((EPW,), jnp.int32),
        pltpu.VMEM((GCH, HW), jnp.float32),
        pltpu.VMEM((GCH, HW), jnp.float32),
        pltpu.VMEM((GCH, HW), jnp.float32),
        pltpu.VMEM((GCH, HW), jnp.float32),
        pltpu.SemaphoreType.DMA,
        pltpu.SemaphoreType.DMA,
        pltpu.SemaphoreType.DMA,
        pltpu.SemaphoreType.DMA,
    ],
)


def _scatter_body(out_e, dsta, zrows, t_out, i0, i1, r0, r1, acc,
                  s0, s1, si0, si1):
    c = lax.axis_index("c")
    s = lax.axis_index("s")
    pltpu.sync_copy(zrows, acc.at[pl.ds(s * ZROWS, ZROWS)])
    plsc.subcore_barrier()

    base = s * SPS
    ibase = c * EDGES_PAD + base
    ibufs = (i0, i1)
    rbufs = (r0, r1)
    rsems = (s0, s1)
    isems = (si0, si1)
    rd = [None] * SCHUNKS
    idd = [None] * SCHUNKS

    def load(j):
        b = j % 2
        rd[j] = pltpu.async_copy(
            out_e.at[pl.ds(base + j * GCH, GCH)], rbufs[b], rsems[b])
        idd[j] = pltpu.async_copy(
            dsta.at[pl.ds(ibase + j * GCH, GCH)], ibufs[b], isems[b])

    load(0)
    for j in range(SCHUNKS):
        rd[j].wait()
        idd[j].wait()
        if j + 1 < SCHUNKS:
            load(j + 1)
        pltpu.sync_copy(rbufs[j % 2], acc.at[ibufs[j % 2]], add=True)
    plsc.subcore_barrier()

    @pl.when(s < 8)
    def _():
        pltpu.sync_copy(acc.at[pl.ds(s * TROWS, TROWS), pl.ds(0, H)],
                        t_out.at[pl.ds(c * HALF + s * TROWS, TROWS)])


_scatter = pl.kernel(
    _scatter_body,
    out_type=jax.ShapeDtypeStruct((N_NODES, H), jnp.float32),
    mesh=_mesh,
    compiler_params=_params,
    scratch_types=[
        pltpu.VMEM((GCH,), jnp.int32),
        pltpu.VMEM((GCH,), jnp.int32),
        pltpu.VMEM((GCH, HW), jnp.float32),
        pltpu.VMEM((GCH, HW), jnp.float32),
        pltpu.VMEM_SHARED((ACC_ROWS, HW), jnp.float32),
        pltpu.SemaphoreType.DMA,
        pltpu.SemaphoreType.DMA,
        pltpu.SemaphoreType.DMA,
        pltpu.SemaphoreType.DMA,
    ],
)

TILE = 1024
GRID = EDGES_PAD // TILE


def _mm_body(pf_ref, l_ref, w_ref, out_ref):
    lv = l_ref[...]
    acc = jnp.dot(lv, w_ref[0], preferred_element_type=jnp.float32)
    for p in range(N_PAIR):
        acc += pf_ref[:, p:p + 1] * jnp.dot(
            lv, w_ref[p + 1], preferred_element_type=jnp.float32)
    out_ref[:, :H] = acc
    out_ref[:, H:] = jnp.zeros((TILE, HW - H), jnp.float32)


_mm = pl.pallas_call(
    _mm_body,
    grid=(GRID,),
    in_specs=[
        pl.BlockSpec((TILE, N_PAIR), lambda i: (i, 0)),
        pl.BlockSpec((TILE, HW), lambda i: (i, 0)),
        pl.BlockSpec((N_PAIR + 1, HW, H), lambda i: (0, 0, 0)),
    ],
    out_specs=pl.BlockSpec((TILE, HW), lambda i: (i, 0)),
    out_shape=jax.ShapeDtypeStruct((EDGES_PAD, HW), jnp.float32),
)


def kernel(pair_features, atom_features, atom_to_pair, W, b):
    pad = EDGES_PAD - N_EDGES
    src_p = jnp.concatenate([atom_to_pair[:, 1], jnp.zeros((pad,), jnp.int32)])
    dst_p = jnp.concatenate(
        [atom_to_pair[:, 0], jnp.full((pad,), N_NODES, jnp.int32)])
    halves = jnp.arange(NC, dtype=jnp.int32)[:, None] * HALF
    dadj = dst_p[None] - halves
    dadj = jnp.where((dadj >= 0) & (dadj < HALF), dadj, DUMMY)
    dadj = dadj.reshape(NC * EDGES_PAD)

    af128 = jnp.pad(atom_features, ((0, 0), (0, HW - H)))
    wbig = jnp.concatenate(
        [b.reshape(1, H, H), W.reshape(N_PAIR, H, H)], axis=0)
    wbig = jnp.pad(wbig.transpose(0, 2, 1), ((0, 0), (0, HW - H), (0, 0)))

    l = _gather(af128, src_p)
    out_e = _mm(pair_features, l, wbig)
    zrows = jnp.zeros((ZROWS, HW), jnp.float32)
    return _scatter(out_e, dadj, zrows)


# speedup vs baseline: 7.2203x; 1.1810x over previous
import jax
import jax.numpy as jnp
from jax import lax
from jax.experimental import pallas as pl
from jax.experimental.pallas import tpu as pltpu
from jax.experimental.pallas import tpu_sc as plsc

N_NODES = 10000
N_EDGES = 50000
N_PAIR = 16
H = 64
HW = 128

NC = 2
NS = 16
NW = NC * NS
EDGES_PAD = 50176
EPW = EDGES_PAD // NW
GCH = 112
CHUNKS = EPW // GCH
NBUF = 4

HALF = N_NODES // NC
ACC_ROWS = 5120
ZROWS = ACC_ROWS // NS
DUMMY = HALF
TROWS = 625
SPS = EDGES_PAD // NS
SCHUNKS = SPS // GCH

_mesh = plsc.VectorSubcoreMesh(core_axis_name="c", subcore_axis_name="s")
_params = pltpu.CompilerParams(use_tc_tiling_on_sc=False)


def _gather_body(af, src, l_out, idx_v, b0, b1, b2, b3, s0, s1, s2, s3):
    c = lax.axis_index("c")
    s = lax.axis_index("s")
    base = (s * NC + c) * EPW
    pltpu.sync_copy(src.at[pl.ds(base, EPW)], idx_v)
    bufs = (b0, b1, b2, b3)
    sems = (s0, s1, s2, s3)

    def g(j):
        return pltpu.async_copy(
            af.at[idx_v.at[pl.ds(j * GCH, GCH)]], bufs[j % NBUF], sems[j % NBUF])

    gd = [None] * CHUNKS
    wd = [None] * CHUNKS
    for j in range(NBUF - 1):
        gd[j] = g(j)
    for j in range(CHUNKS):
        gd[j].wait()
        if j > 0:
            wd[j - 1].wait()
        wd[j] = pltpu.async_copy(
            bufs[j % NBUF], l_out.at[pl.ds(base + j * GCH, GCH)], sems[j % NBUF])
        if j + NBUF - 1 < CHUNKS:
            gd[j + NBUF - 1] = g(j + NBUF - 1)
    wd[CHUNKS - 1].wait()


_gather = pl.kernel(
    _gather_body,
    out_type=jax.ShapeDtypeStruct((EDGES_PAD, HW), jnp.float32),
    mesh=_mesh,
    compiler_params=_params,
    scratch_types=[
        pltpu.VMEM((EPW,), jnp.int32),
        pltpu.VMEM((GCH, HW), jnp.float32),
        pltpu.VMEM((GCH, HW), jnp.float32),
        pltpu.VMEM((GCH, HW), jnp.float32),
        pltpu.VMEM((GCH, HW), jnp.float32),
        pltpu.SemaphoreType.DMA,
        pltpu.SemaphoreType.DMA,
        pltpu.SemaphoreType.DMA,
        pltpu.SemaphoreType.DMA,
    ],
)


def _scatter_body(out_e, dsta, zrows, t_out, i0, i1, r0, r1, acc,
                  s0, s1, si0, si1):
    c = lax.axis_index("c")
    s = lax.axis_index("s")
    pltpu.sync_copy(zrows, acc.at[pl.ds(s * ZROWS, ZROWS)])
    plsc.subcore_barrier()

    base = s * SPS
    ibase = c * EDGES_PAD + base
    ibufs = (i0, i1)
    rbufs = (r0, r1)
    rsems = (s0, s1)
    isems = (si0, si1)
    rd = [None] * SCHUNKS
    idd = [None] * SCHUNKS

    def load(j):
        b = j % 2
        rd[j] = pltpu.async_copy(
            out_e.at[pl.ds(base + j * GCH, GCH), pl.ds(0, H)], rbufs[b], rsems[b])
        idd[j] = pltpu.async_copy(
            dsta.at[pl.ds(ibase + j * GCH, GCH)], ibufs[b], isems[b])

    load(0)
    for j in range(SCHUNKS):
        rd[j].wait()
        idd[j].wait()
        if j + 1 < SCHUNKS:
            load(j + 1)
        pltpu.sync_copy(rbufs[j % 2], acc.at[ibufs[j % 2]], add=True)
    plsc.subcore_barrier()

    @pl.when(s < 8)
    def _():
        pltpu.sync_copy(acc.at[pl.ds(s * TROWS, TROWS)],
                        t_out.at[pl.ds(c * HALF + s * TROWS, TROWS)])


_scatter = pl.kernel(
    _scatter_body,
    out_type=jax.ShapeDtypeStruct((N_NODES, H), jnp.float32),
    mesh=_mesh,
    compiler_params=_params,
    scratch_types=[
        pltpu.VMEM((GCH,), jnp.int32),
        pltpu.VMEM((GCH,), jnp.int32),
        pltpu.VMEM((GCH, H), jnp.float32),
        pltpu.VMEM((GCH, H), jnp.float32),
        pltpu.VMEM_SHARED((ACC_ROWS, H), jnp.float32),
        pltpu.SemaphoreType.DMA,
        pltpu.SemaphoreType.DMA,
        pltpu.SemaphoreType.DMA,
        pltpu.SemaphoreType.DMA,
    ],
)

TILE = 2048
GRID = EDGES_PAD // TILE if EDGES_PAD % TILE == 0 else EDGES_PAD // TILE + 1


def _mm_body(pf_ref, l_ref, w_ref, out_ref):
    lv = l_ref[...].astype(jnp.bfloat16)
    pfb = pf_ref[...].astype(jnp.bfloat16)
    acc = jnp.dot(lv, w_ref[0], preferred_element_type=jnp.float32)
    for p in range(N_PAIR):
        acc += jnp.dot(pfb[:, p:p + 1] * lv, w_ref[p + 1],
                       preferred_element_type=jnp.float32)
    out_ref[:, :H] = acc


_mm = pl.pallas_call(
    _mm_body,
    grid=(GRID,),
    in_specs=[
        pl.BlockSpec((TILE, N_PAIR), lambda i: (i, 0)),
        pl.BlockSpec((TILE, HW), lambda i: (i, 0)),
        pl.BlockSpec((N_PAIR + 1, HW, H), lambda i: (0, 0, 0)),
    ],
    out_specs=pl.BlockSpec((TILE, HW), lambda i: (i, 0)),
    out_shape=jax.ShapeDtypeStruct((EDGES_PAD, HW), jnp.float32),
)


def kernel(pair_features, atom_features, atom_to_pair, W, b):
    pad = EDGES_PAD - N_EDGES
    src_p = jnp.concatenate([atom_to_pair[:, 1], jnp.zeros((pad,), jnp.int32)])
    dst_p = jnp.concatenate(
        [atom_to_pair[:, 0], jnp.full((pad,), N_NODES, jnp.int32)])
    halves = jnp.arange(NC, dtype=jnp.int32)[:, None] * HALF
    dadj = dst_p[None] - halves
    dadj = jnp.where((dadj >= 0) & (dadj < HALF), dadj, DUMMY)
    dadj = dadj.reshape(NC * EDGES_PAD)

    af128 = jnp.pad(atom_features, ((0, 0), (0, HW - H)))
    wbig = jnp.concatenate(
        [b.reshape(1, H, H), W.reshape(N_PAIR, H, H)], axis=0)
    wbig = jnp.pad(wbig.transpose(0, 2, 1), ((0, 0), (0, HW - H), (0, 0)))
    wbig = wbig.astype(jnp.bfloat16)

    l = _gather(af128, src_p)
    out_e = _mm(pair_features, l, wbig)
    zrows = jnp.zeros((ZROWS, H), jnp.float32)
    return _scatter(out_e, dadj, zrows)
